# baseline (speedup 1.0000x reference)
import math

import jax
import jax.numpy as jnp
from jax import lax
from jax.experimental import pallas as pl
from jax.experimental.pallas import tpu as pltpu
from jax.experimental.pallas import tpu_sc as plsc

B = 4096
F = 26
NCAT = 24
NNUM = F - NCAT
C = 16
V = 100

NC = 2
NS = 16
NW = NC * NS
BPW = B // NW
NGROUPS = BPW // 16

CAT_WORDS = C * NCAT * V
G_OFF = CAT_WORDS
CP_OFF = G_OFF + NNUM * V * C
TBL_TOTAL = CP_OFF + C


def _nbc_body(x, cat_tbl, cp, means, stds, out, tbl_v, x_v, ms_v, ss_v,
              out_v, sem):
    wid = lax.axis_index("s") * NC + lax.axis_index("c")
    base = wid * BPW

    tbl_dma = pltpu.async_copy(cat_tbl, tbl_v.at[pl.ds(0, CAT_WORDS)], sem)
    pltpu.sync_copy(x.at[pl.ds(base, BPW), :], x_v)
    pltpu.sync_copy(cp, tbl_v.at[pl.ds(CP_OFF, C)])
    pltpu.sync_copy(means, ms_v)
    pltpu.sync_copy(stds, ss_v)

    lane2 = jnp.arange(16, dtype=jnp.int32) * NNUM
    for jn in range(NNUM):
        m = plsc.load_gather(ms_v, [lane2 + jn])
        s = plsc.load_gather(ss_v, [lane2 + jn])
        coef = 1.0 / (2.0 * math.pi * (s * s))
        for v in range(V):
            t = (float(v) - m) / s
            row = coef * jnp.exp(-0.5 * (t * t))
            tbl_v[pl.ds(G_OFF + (jn * V + v) * C, C)] = row

    tbl_dma.wait()

    cps = [plsc.load_gather(tbl_v, [jnp.full((16,), CP_OFF + i, jnp.int32)])
           for i in range(C)]
    lanes = jnp.arange(16, dtype=jnp.int32)

    @plsc.parallel_loop(0, NGROUPS, unroll=2)
    def _group(g):
        rows = lanes + g * 16
        accs = list(cps)
        for j in range(NCAT):
            vj = plsc.load_gather(x_v, [rows, jnp.full((16,), j, jnp.int32)])
            for i in range(C):
                factor = plsc.load_gather(tbl_v, [vj + (i * NCAT * V + j * V)])
                accs[i] = accs[i] * factor
        for jn in range(NNUM):
            vj = plsc.load_gather(
                x_v, [rows, jnp.full((16,), NCAT + jn, jnp.int32)])
            vj16 = vj * C
            for i in range(C):
                factor = plsc.load_gather(
                    tbl_v, [vj16 + (G_OFF + jn * V * C + i)])
                accs[i] = accs[i] * factor
        best = accs[0]
        besti = jnp.zeros((16,), jnp.int32)
        for i in range(1, C):
            gt = accs[i] > best
            best = jnp.where(gt, accs[i], best)
            besti = jnp.where(gt, jnp.full((16,), i, jnp.int32), besti)
        out_v[pl.ds(g * 16, 16)] = besti

    pltpu.sync_copy(out_v, out.at[pl.ds(base, BPW)])


def _nbc_sc(x, cat_tbl, class_probs, means_flat, stds_flat):
    mesh = plsc.VectorSubcoreMesh(core_axis_name="c", subcore_axis_name="s")
    return pl.kernel(
        _nbc_body,
        out_type=jax.ShapeDtypeStruct((B,), jnp.int32),
        mesh=mesh,
        compiler_params=pltpu.CompilerParams(needs_layout_passes=False),
        scratch_types=[
            pltpu.VMEM((TBL_TOTAL,), jnp.float32),
            pltpu.VMEM((BPW, F), jnp.int32),
            pltpu.VMEM((C * NNUM,), jnp.float32),
            pltpu.VMEM((C * NNUM,), jnp.float32),
            pltpu.VMEM((BPW,), jnp.int32),
            pltpu.SemaphoreType.DMA,
        ],
    )(x, cat_tbl, class_probs, means_flat, stds_flat)


def kernel(X, class_probs, cond_probs_cat, means, stds):
    return _nbc_sc(X, cond_probs_cat.reshape(-1), class_probs,
                   means.reshape(-1), stds.reshape(-1))


# speedup vs baseline: 1.1754x; 1.1754x over previous
import math

import jax
import jax.numpy as jnp
from jax import lax
from jax.experimental import pallas as pl
from jax.experimental.pallas import tpu as pltpu
from jax.experimental.pallas import tpu_sc as plsc

B = 4096
F = 26
NCAT = 24
NNUM = F - NCAT
C = 16
V = 100

NC = 2
NS = 16
NW = NC * NS
BPW = B // NW
NGROUPS = BPW // 16

CAT_WORDS = C * NCAT * V
G_OFF = CAT_WORDS
CP_OFF = G_OFF + NNUM * V * C
TBL_TOTAL = CP_OFF + C


def _nbc_body(x, cat_tbl, cp, means, stds, out, tbl_v, x_v, ms_v, ss_v,
              out_v, sem):
    wid = lax.axis_index("s") * NC + lax.axis_index("c")
    base = wid * BPW

    tbl_dma = pltpu.async_copy(cat_tbl, tbl_v.at[pl.ds(0, CAT_WORDS)], sem)
    pltpu.sync_copy(x.at[pl.ds(base, BPW), :], x_v)
    pltpu.sync_copy(cp, tbl_v.at[pl.ds(CP_OFF, C)])
    pltpu.sync_copy(means, ms_v)
    pltpu.sync_copy(stds, ss_v)

    lane2 = jnp.arange(16, dtype=jnp.int32) * NNUM
    for jn in range(NNUM):
        m = plsc.load_gather(ms_v, [lane2 + jn])
        s = plsc.load_gather(ss_v, [lane2 + jn])
        coef = 1.0 / (2.0 * math.pi * (s * s))
        for v in range(V):
            t = (float(v) - m) / s
            row = coef * jnp.exp(-0.5 * (t * t))
            tbl_v[pl.ds(G_OFF + (jn * V + v) * C, C)] = row

    tbl_dma.wait()

    cps = [plsc.load_gather(tbl_v, [jnp.full((16,), CP_OFF + i, jnp.int32)])
           for i in range(C)]
    lanes = jnp.arange(16, dtype=jnp.int32)

    @plsc.parallel_loop(0, NGROUPS)
    def _group(g):
        rows = lanes + g * 16
        accs = list(cps)
        for j in range(NCAT):
            vj = plsc.load_gather(x_v, [rows, jnp.full((16,), j, jnp.int32)])
            for i in range(C):
                factor = plsc.load_gather(tbl_v, [vj + (i * NCAT * V + j * V)])
                accs[i] = accs[i] * factor
        for jn in range(NNUM):
            vj = plsc.load_gather(
                x_v, [rows, jnp.full((16,), NCAT + jn, jnp.int32)])
            vj16 = vj * C
            for i in range(C):
                factor = plsc.load_gather(
                    tbl_v, [vj16 + (G_OFF + jn * V * C + i)])
                accs[i] = accs[i] * factor
        best = accs[0]
        besti = jnp.zeros((16,), jnp.int32)
        for i in range(1, C):
            gt = accs[i] > best
            best = jnp.where(gt, accs[i], best)
            besti = jnp.where(gt, jnp.full((16,), i, jnp.int32), besti)
        out_v[pl.ds(g * 16, 16)] = besti

    pltpu.sync_copy(out_v, out.at[pl.ds(base, BPW)])


def _nbc_sc(x, cat_tbl, class_probs, means_flat, stds_flat):
    mesh = plsc.VectorSubcoreMesh(core_axis_name="c", subcore_axis_name="s")
    return pl.kernel(
        _nbc_body,
        out_type=jax.ShapeDtypeStruct((B,), jnp.int32),
        mesh=mesh,
        compiler_params=pltpu.CompilerParams(needs_layout_passes=False),
        scratch_types=[
            pltpu.VMEM((TBL_TOTAL,), jnp.float32),
            pltpu.VMEM((BPW, F), jnp.int32),
            pltpu.VMEM((C * NNUM,), jnp.float32),
            pltpu.VMEM((C * NNUM,), jnp.float32),
            pltpu.VMEM((BPW,), jnp.int32),
            pltpu.SemaphoreType.DMA,
        ],
    )(x, cat_tbl, class_probs, means_flat, stds_flat)


def kernel(X, class_probs, cond_probs_cat, means, stds):
    return _nbc_sc(X, cond_probs_cat.reshape(-1), class_probs,
                   means.reshape(-1), stds.reshape(-1))
